# Initial kernel scaffold; baseline (speedup 1.0000x reference)
import jax
import jax.numpy as jnp
from jax.experimental import pallas as pl


def kernel(x1_batch, x2_batch, WA0, WB0, WC0, WD0, WE0, WA1, WB1, WC1, WD1, WE1, Ws, bs, Wid, bid):
    raise NotImplementedError("write your pallas kernel here")


# speedup vs baseline: 6.0217x; 6.0217x over previous
import jax
import jax.numpy as jnp
from jax.experimental import pallas as pl

_T = 512


def _dot(a, b):
    return jax.lax.dot(a, b, precision=jax.lax.Precision.HIGHEST)


def _gnn_body(x1_ref, x2_ref, w0_ref, we0_ref, w1_ref, we1_ref, fp_ref, fc_ref):
    t = pl.program_id(1)
    nt = pl.num_programs(1)
    xa = x1_ref[0]
    xb = x2_ref[0]
    we0 = we0_ref[...]

    pa = _dot(xa, w0_ref[...])
    pb = _dot(xb, w0_ref[...])
    aa, ba, ca, da = pa[:, :128], pa[:, 128:256], pa[:, 256:384], pa[:, 384:]
    ab, bb, cb, db = pb[:, :128], pb[:, 128:256], pb[:, 256:384], pb[:, 384:]
    e_ab = ca + db + we0
    e_ba = cb + da + we0
    ha = jax.nn.relu(aa + jax.nn.sigmoid(e_ba) * bb)
    hb = jax.nn.relu(ab + jax.nn.sigmoid(e_ab) * ba)

    qa = _dot(ha, w1_ref[...])
    qb = _dot(hb, w1_ref[...])
    ea = _dot(e_ab, we1_ref[...])
    eb = _dot(e_ba, we1_ref[...])
    e2_ab = qa[:, 64:96] + qb[:, 96:128] + ea
    e2_ba = qb[:, 64:96] + qa[:, 96:128] + eb
    h2a = jax.nn.relu(qa[:, :32] + jax.nn.sigmoid(e2_ba) * qb[:, 32:64])
    h2b = jax.nn.relu(qb[:, :32] + jax.nn.sigmoid(e2_ab) * qa[:, 32:64])

    inv = jnp.float32(1.0) / jnp.float32(_T * nt)
    fp_part = jnp.concatenate([ha.sum(0), h2a.sum(0)])[None, None, :] * inv
    fc_part = jnp.concatenate([hb.sum(0), h2b.sum(0)])[None, None, :] * inv

    @pl.when(t == 0)
    def _init():
        fp_ref[...] = fp_part
        fc_ref[...] = fc_part

    @pl.when(t != 0)
    def _acc():
        fp_ref[...] += fp_part
        fc_ref[...] += fc_part


def _head_body(fp_ref, fc_ref, ws_ref, bs_ref, wid_ref, bid_ref,
               y_ref, p0_ref, p1_ref, p2_ref, p3_ref, c_ref):
    fp = fp_ref[...]
    fc = fc_ref[...]
    c_ref[...] = 0.5 * (fp + fc)
    y_ref[...] = _dot(jnp.abs(fp - fc), ws_ref[...]) + bs_ref[...]
    f = jnp.concatenate([fp, fc], axis=0)
    for i, p_ref in enumerate((p0_ref, p1_ref, p2_ref, p3_ref)):
        p_ref[...] = _dot(f, wid_ref[i]) + bid_ref[i][None, :]


def kernel(x1_batch, x2_batch, WA0, WB0, WC0, WD0, WE0,
           WA1, WB1, WC1, WD1, WE1, Ws, bs, Wid, bid):
    B, L, d = x1_batch.shape
    d1 = WA1.shape[1]
    feat = d + d1
    out_dim = Ws.shape[1]

    w0 = jnp.concatenate([WA0, WB0, WC0, WD0], axis=1)
    w1 = jnp.concatenate([WA1, WB1, WC1, WD1], axis=1)

    fp, fc = pl.pallas_call(
        _gnn_body,
        grid=(B, L // _T),
        in_specs=[
            pl.BlockSpec((1, _T, d), lambda b, t: (b, t, 0)),
            pl.BlockSpec((1, _T, d), lambda b, t: (b, t, 0)),
            pl.BlockSpec((d, 4 * d), lambda b, t: (0, 0)),
            pl.BlockSpec((1, d), lambda b, t: (0, 0)),
            pl.BlockSpec((d, d), lambda b, t: (0, 0)),
            pl.BlockSpec((d, d1), lambda b, t: (0, 0)),
        ],
        out_specs=[
            pl.BlockSpec((1, 1, feat), lambda b, t: (b, 0, 0)),
            pl.BlockSpec((1, 1, feat), lambda b, t: (b, 0, 0)),
        ],
        out_shape=[
            jax.ShapeDtypeStruct((B, 1, feat), jnp.float32),
            jax.ShapeDtypeStruct((B, 1, feat), jnp.float32),
        ],
    )(x1_batch, x2_batch, w0, WE0, w1, WE1)
    fp = fp.reshape(B, feat)
    fc = fc.reshape(B, feat)

    nj = Wid.shape[1]
    rows = 4 * jnp.arange(nj)[None, :] + jnp.arange(4)[:, None]
    wid_e = jnp.zeros((4, feat, out_dim), jnp.float32).at[
        jnp.arange(4)[:, None], rows].set(Wid)

    y, p0, p1, p2, p3, center = pl.pallas_call(
        _head_body,
        out_shape=[
            jax.ShapeDtypeStruct((B, out_dim), jnp.float32),
            jax.ShapeDtypeStruct((2 * B, out_dim), jnp.float32),
            jax.ShapeDtypeStruct((2 * B, out_dim), jnp.float32),
            jax.ShapeDtypeStruct((2 * B, out_dim), jnp.float32),
            jax.ShapeDtypeStruct((2 * B, out_dim), jnp.float32),
            jax.ShapeDtypeStruct((B, feat), jnp.float32),
        ],
    )(fp, fc, Ws, bs.reshape(1, -1), wid_e, bid)

    return (y, fp, fc, p0, p1, p2, p3, center)


# speedup vs baseline: 15.4927x; 2.5728x over previous
import jax
import jax.numpy as jnp
from jax.experimental import pallas as pl

_T = 512


def _dot(a, b):
    return jax.lax.dot(a, b, precision=jax.lax.Precision.DEFAULT)


def _gnn_body(x1_ref, x2_ref, w0_ref, we0_ref, w1_ref, we1_ref, fp_ref, fc_ref):
    t = pl.program_id(1)
    nt = pl.num_programs(1)
    xa = x1_ref[0]
    xb = x2_ref[0]
    we0 = we0_ref[...]

    pa = _dot(xa, w0_ref[...])
    pb = _dot(xb, w0_ref[...])
    aa, ba, ca, da = pa[:, :128], pa[:, 128:256], pa[:, 256:384], pa[:, 384:]
    ab, bb, cb, db = pb[:, :128], pb[:, 128:256], pb[:, 256:384], pb[:, 384:]
    e_ab = ca + db + we0
    e_ba = cb + da + we0
    ha = jax.nn.relu(aa + jax.nn.sigmoid(e_ba) * bb)
    hb = jax.nn.relu(ab + jax.nn.sigmoid(e_ab) * ba)

    qa = _dot(ha, w1_ref[...])
    qb = _dot(hb, w1_ref[...])
    ea = _dot(e_ab, we1_ref[...])
    eb = _dot(e_ba, we1_ref[...])
    e2_ab = qa[:, 64:96] + qb[:, 96:128] + ea
    e2_ba = qb[:, 64:96] + qa[:, 96:128] + eb
    h2a = jax.nn.relu(qa[:, :32] + jax.nn.sigmoid(e2_ba) * qb[:, 32:64])
    h2b = jax.nn.relu(qb[:, :32] + jax.nn.sigmoid(e2_ab) * qa[:, 32:64])

    inv = jnp.float32(1.0) / jnp.float32(_T * nt)
    fp_part = jnp.concatenate([ha.sum(0), h2a.sum(0)])[None, None, :] * inv
    fc_part = jnp.concatenate([hb.sum(0), h2b.sum(0)])[None, None, :] * inv

    @pl.when(t == 0)
    def _init():
        fp_ref[...] = fp_part
        fc_ref[...] = fc_part

    @pl.when(t != 0)
    def _acc():
        fp_ref[...] += fp_part
        fc_ref[...] += fc_part


def _head_body(fp_ref, fc_ref, ws_ref, bs_ref, wid_ref, bid_ref,
               y_ref, p0_ref, p1_ref, p2_ref, p3_ref, c_ref):
    fp = fp_ref[...]
    fc = fc_ref[...]
    c_ref[...] = 0.5 * (fp + fc)
    y_ref[...] = _dot(jnp.abs(fp - fc), ws_ref[...]) + bs_ref[...]
    f = jnp.concatenate([fp, fc], axis=0)
    for i, p_ref in enumerate((p0_ref, p1_ref, p2_ref, p3_ref)):
        p_ref[...] = _dot(f, wid_ref[i]) + bid_ref[i][None, :]


def kernel(x1_batch, x2_batch, WA0, WB0, WC0, WD0, WE0,
           WA1, WB1, WC1, WD1, WE1, Ws, bs, Wid, bid):
    B, L, d = x1_batch.shape
    d1 = WA1.shape[1]
    feat = d + d1
    out_dim = Ws.shape[1]

    w0 = jnp.concatenate([WA0, WB0, WC0, WD0], axis=1)
    w1 = jnp.concatenate([WA1, WB1, WC1, WD1], axis=1)

    fp, fc = pl.pallas_call(
        _gnn_body,
        grid=(B, L // _T),
        in_specs=[
            pl.BlockSpec((1, _T, d), lambda b, t: (b, t, 0)),
            pl.BlockSpec((1, _T, d), lambda b, t: (b, t, 0)),
            pl.BlockSpec((d, 4 * d), lambda b, t: (0, 0)),
            pl.BlockSpec((1, d), lambda b, t: (0, 0)),
            pl.BlockSpec((d, d), lambda b, t: (0, 0)),
            pl.BlockSpec((d, d1), lambda b, t: (0, 0)),
        ],
        out_specs=[
            pl.BlockSpec((1, 1, feat), lambda b, t: (b, 0, 0)),
            pl.BlockSpec((1, 1, feat), lambda b, t: (b, 0, 0)),
        ],
        out_shape=[
            jax.ShapeDtypeStruct((B, 1, feat), jnp.float32),
            jax.ShapeDtypeStruct((B, 1, feat), jnp.float32),
        ],
    )(x1_batch, x2_batch, w0, WE0, w1, WE1)
    fp = fp.reshape(B, feat)
    fc = fc.reshape(B, feat)

    nj = Wid.shape[1]
    rows = 4 * jnp.arange(nj)[None, :] + jnp.arange(4)[:, None]
    wid_e = jnp.zeros((4, feat, out_dim), jnp.float32).at[
        jnp.arange(4)[:, None], rows].set(Wid)

    y, p0, p1, p2, p3, center = pl.pallas_call(
        _head_body,
        out_shape=[
            jax.ShapeDtypeStruct((B, out_dim), jnp.float32),
            jax.ShapeDtypeStruct((2 * B, out_dim), jnp.float32),
            jax.ShapeDtypeStruct((2 * B, out_dim), jnp.float32),
            jax.ShapeDtypeStruct((2 * B, out_dim), jnp.float32),
            jax.ShapeDtypeStruct((2 * B, out_dim), jnp.float32),
            jax.ShapeDtypeStruct((B, feat), jnp.float32),
        ],
    )(fp, fc, Ws, bs.reshape(1, -1), wid_e, bid)

    return (y, fp, fc, p0, p1, p2, p3, center)
